# baseline (speedup 1.0000x reference)
import functools

import jax
import jax.numpy as jnp
from jax import lax
from jax.experimental import pallas as pl
from jax.experimental.pallas import tpu as pltpu
from jax.experimental.pallas import tpu_sc as plsc

KK = 256
NROWS = 128
NCOLS = 32768
NC, NS, L = 2, 16, 16
NW = NC * NS
RPW = NROWS // NW
NV = NCOLS // L
NG = NV // L
NB = 256
CCAP = NCOLS + 16
WCAP = KK + 16

_mesh = plsc.VectorSubcoreMesh(core_axis_name="c", subcore_axis_name="s",
                               num_cores=NC, num_subcores=NS)


def _sc_body(sim_hbm, out_hbm, row_v, cidx_v, hist_v, pfx_v, win_v):
    iota = lax.iota(jnp.int32, L)
    ones = jnp.ones((L,), jnp.int32)
    zeros = jnp.zeros((L,), jnp.int32)
    zerosf = jnp.zeros((L,), jnp.float32)

    def tokey(v):
        b = lax.bitcast_convert_type(v, jnp.uint32)
        return jnp.where(b >= jnp.uint32(0x80000000), ~b,
                         b | jnp.uint32(0x80000000))

    def keyat(ix):
        return tokey(plsc.load_gather(row_v, [ix & jnp.int32(NCOLS - 1)]))

    def scalar(x):
        return jnp.max(x) if x.ndim else x

    def hist_full():
        for g in range(L):
            hist_v[pl.ds(g * L, L)] = zeros

        @plsc.parallel_loop(0, NV, 1, unroll=8)
        def _hist(i):
            k = tokey(row_v[pl.ds(i * L, L)])
            d = (k >> jnp.uint32(24)).astype(jnp.int32)
            cnts, last = plsc.scan_count(d)
            plsc.addupdate_scatter(hist_v, [d], cnts, mask=last)

    def select8(k_rem):
        def gbody(gr, st):
            carry, found, digit, above = st
            g = jnp.int32(L - 1) - gr
            M = hist_v[pl.ds(g * L, L)]
            revg = lax.rev(M, (0,))
            rcg = plsc.cumsum(revg) + carry
            ge = rcg >= k_rem
            anyge = jnp.max(ge.astype(jnp.int32))
            istar = scalar(plsc.all_reduce_ffs(ge))
            selrc = jnp.max(jnp.where(iota == istar, rcg, 0))
            selbin = jnp.max(jnp.where(iota == istar, revg, 0))
            hit = (anyge == 1) & (found == 0)
            digit = jnp.where(hit, g * L + (jnp.int32(L - 1) - istar), digit)
            above = jnp.where(hit, selrc - selbin, above)
            found = jnp.where(anyge == 1, jnp.int32(1), found)
            return (rcg[L - 1], found, digit, above)

        st = (jnp.int32(0), jnp.int32(0), jnp.int32(0), jnp.int32(0))
        _, _, digit, above = lax.fori_loop(0, L, gbody, st)
        return digit, k_rem - above

    def compact(prefix):
        @plsc.parallel_loop(0, NG, 1, unroll=2)
        def _pa(i):
            acc = zeros
            for t in range(L):
                k = tokey(row_v[pl.ds((i * L + t) * L, L)])
                m = (k >> jnp.uint32(24)) == prefix
                c = plsc.all_reduce_population_count(m)
                acc = jnp.where(iota == t, c, acc)
            pfx_v[pl.ds(i * L, L)] = acc

        def _pb(i, carry):
            c = pfx_v[pl.ds(i * L, L)]
            inc = plsc.cumsum(c)
            pfx_v[pl.ds(i * L, L)] = inc - c + carry
            return carry + inc[L - 1]

        n = lax.fori_loop(0, NG, _pb, jnp.int32(0))

        @plsc.parallel_loop(0, NV, 1, unroll=8)
        def _pc(i):
            k = tokey(row_v[pl.ds(i * L, L)])
            m = (k >> jnp.uint32(24)) == prefix
            mi = m.astype(jnp.int32)
            base = pfx_v[pl.ds(i, L)][0]
            dest = plsc.cumsum(mi) - mi + base
            plsc.store_scatter(cidx_v, [dest], i * L + iota, mask=m)

        return n

    def hist_cand(n, prefix, sp, sd):
        hist_v[pl.ds(0, L)] = zeros
        trips = (n + L - 1) // L

        @plsc.parallel_loop(0, trips, 1, unroll=2)
        def _hc(i):
            k = keyat(cidx_v[pl.ds(i * L, L)])
            act = ((i * L + iota) < n) & ((k >> jnp.uint32(sp)) == prefix)
            d = ((k >> jnp.uint32(sd)) & jnp.uint32(15)).astype(jnp.int32)
            cnts, last = plsc.scan_count(d, act)
            plsc.addupdate_scatter(hist_v, [d], cnts, mask=last & act)

    def select4(k_rem):
        M = hist_v[pl.ds(0, L)]
        rev = lax.rev(M, (0,))
        rc = plsc.cumsum(rev)
        istar = scalar(plsc.all_reduce_ffs(rc >= k_rem))
        sel = jnp.max(jnp.where(iota == istar, rc, 0))
        bincnt = jnp.max(jnp.where(iota == istar, rev, 0))
        digit = jnp.int32(L - 1) - istar
        return digit, k_rem - (sel - bincnt)

    wid = lax.axis_index("s") * NC + lax.axis_index("c")

    def row_body(j, carry):
        r = wid * RPW + j
        pltpu.sync_copy(sim_hbm.at[r], row_v)

        hist_full()
        d1, k_rem = select8(jnp.int32(KK))
        prefix = d1.astype(jnp.uint32)
        n = compact(prefix)

        for lvl in range(6):
            sd = 20 - 4 * lvl
            hist_cand(n, prefix, sd + 4, sd)
            dl, k_rem = select4(k_rem)
            prefix = (prefix << jnp.uint32(4)) | dl.astype(jnp.uint32)

        T = prefix
        m_take = k_rem

        trips = (n + L - 1) // L

        def wbody(i, st):
            cnt_eq, cnt_w = st
            ix = cidx_v[pl.ds(i * L, L)]
            k = keyat(ix)
            eq = ((i * L + iota) < n) & (k == T)
            eqi = eq.astype(jnp.int32)
            excl = plsc.cumsum(eqi) - eqi + cnt_eq
            win = eq & (excl < m_take)
            plsc.store_compressed(win_v.at[pl.ds(cnt_w, L)], ix, mask=win)
            cnt_eq = cnt_eq + scalar(plsc.all_reduce_population_count(eq))
            cnt_w = cnt_w + scalar(plsc.all_reduce_population_count(win))
            return (cnt_eq, cnt_w)

        lax.fori_loop(0, trips, wbody, (jnp.int32(0), jnp.int32(0)))

        @plsc.parallel_loop(0, NV, 1, unroll=8)
        def _mask(i):
            k = tokey(row_v[pl.ds(i * L, L)])
            row_v[pl.ds(i * L, L)] = jnp.where(
                k > T, jnp.float32(0), jnp.float32(jnp.inf))

        wtrips = (m_take + L - 1) // L

        def pbody(i, c):
            ixw = win_v[pl.ds(i * L, L)] & jnp.int32(NCOLS - 1)
            lanem = (i * L + iota) < m_take
            plsc.store_scatter(row_v, [ixw], zerosf, mask=lanem)
            return c

        lax.fori_loop(0, wtrips, pbody, jnp.int32(0))

        pltpu.sync_copy(row_v, out_hbm.at[r])
        return carry

    lax.fori_loop(0, RPW, row_body, jnp.int32(0))


_sc_kernel = functools.partial(
    pl.kernel,
    out_type=jax.ShapeDtypeStruct((NROWS, NCOLS), jnp.float32),
    mesh=_mesh,
    compiler_params=pltpu.CompilerParams(needs_layout_passes=False),
    scratch_types=[
        pltpu.VMEM((NCOLS,), jnp.float32),
        pltpu.VMEM((CCAP,), jnp.int32),
        pltpu.VMEM((NB,), jnp.int32),
        pltpu.VMEM((NV + L,), jnp.int32),
        pltpu.VMEM((WCAP,), jnp.int32),
    ],
)(_sc_body)


def kernel(sim):
    return _sc_kernel(sim)


# speedup vs baseline: 1.0010x; 1.0010x over previous
import functools

import jax
import jax.numpy as jnp
from jax import lax
from jax.experimental import pallas as pl
from jax.experimental.pallas import tpu as pltpu
from jax.experimental.pallas import tpu_sc as plsc

KK = 256
NROWS = 128
NCOLS = 32768
NC, NS, L = 2, 16, 16
NW = NC * NS
RPW = NROWS // NW
NV = NCOLS // L
NG = NV // L
NB = 256
CCAP = NCOLS + 16
WCAP = KK + 16

_mesh = plsc.VectorSubcoreMesh(core_axis_name="c", subcore_axis_name="s",
                               num_cores=NC, num_subcores=NS)


def _sc_body(sim_hbm, out_hbm, row_v, cidx_v, hist_v, pfx_v, win_v):
    iota = lax.iota(jnp.int32, L)
    ones = jnp.ones((L,), jnp.int32)
    zeros = jnp.zeros((L,), jnp.int32)
    zerosf = jnp.zeros((L,), jnp.float32)

    def tokey(v):
        b = lax.bitcast_convert_type(v, jnp.uint32)
        return jnp.where(b >= jnp.uint32(0x80000000), ~b,
                         b | jnp.uint32(0x80000000))

    def keyat(ix):
        return tokey(plsc.load_gather(row_v, [ix & jnp.int32(NCOLS - 1)]))

    def scalar(x):
        return jnp.max(x) if x.ndim else x

    def hist_full():
        for g in range(L):
            hist_v[pl.ds(g * L, L)] = zeros

        @plsc.parallel_loop(0, NV, 1, unroll=8)
        def _hist(i):
            k = tokey(row_v[pl.ds(i * L, L)])
            d = (k >> jnp.uint32(24)).astype(jnp.int32)
            cnts, last = plsc.scan_count(d)
            plsc.addupdate_scatter(hist_v, [d], cnts, mask=last)

    def select8(k_rem):
        def gbody(gr, st):
            carry, found, digit, above = st
            g = jnp.int32(L - 1) - gr
            M = hist_v[pl.ds(g * L, L)]
            revg = lax.rev(M, (0,))
            rcg = plsc.cumsum(revg) + carry
            ge = rcg >= k_rem
            anyge = jnp.max(ge.astype(jnp.int32))
            istar = scalar(plsc.all_reduce_ffs(ge))
            selrc = jnp.max(jnp.where(iota == istar, rcg, 0))
            selbin = jnp.max(jnp.where(iota == istar, revg, 0))
            hit = (anyge == 1) & (found == 0)
            digit = jnp.where(hit, g * L + (jnp.int32(L - 1) - istar), digit)
            above = jnp.where(hit, selrc - selbin, above)
            found = jnp.where(anyge == 1, jnp.int32(1), found)
            return (rcg[L - 1], found, digit, above)

        st = (jnp.int32(0), jnp.int32(0), jnp.int32(0), jnp.int32(0))
        _, _, digit, above = lax.fori_loop(0, L, gbody, st)
        return digit, k_rem - above

    def compact(prefix):
        @plsc.parallel_loop(0, NG, 1, unroll=2)
        def _pa(i):
            acc = zeros
            for t in range(L):
                k = tokey(row_v[pl.ds((i * L + t) * L, L)])
                m = (k >> jnp.uint32(24)) == prefix
                c = plsc.all_reduce_population_count(m)
                acc = jnp.where(iota == t, c, acc)
            pfx_v[pl.ds(i * L, L)] = acc

        def _pb(i, carry):
            c = pfx_v[pl.ds(i * L, L)]
            inc = plsc.cumsum(c)
            pfx_v[pl.ds(i * L, L)] = inc - c + carry
            return carry + inc[L - 1]

        n = lax.fori_loop(0, NG, _pb, jnp.int32(0))

        @plsc.parallel_loop(0, NV, 1, unroll=8)
        def _pc(i):
            k = tokey(row_v[pl.ds(i * L, L)])
            m = (k >> jnp.uint32(24)) == prefix
            base = pfx_v[pl.ds(i, L)][0]
            plsc.store_compressed(cidx_v.at[pl.ds(base, L)], i * L + iota,
                                  mask=m)

        return n

    def hist_cand(n, prefix, sp, sd):
        hist_v[pl.ds(0, L)] = zeros
        trips = (n + L - 1) // L

        @plsc.parallel_loop(0, trips, 1, unroll=2)
        def _hc(i):
            k = keyat(cidx_v[pl.ds(i * L, L)])
            act = ((i * L + iota) < n) & ((k >> jnp.uint32(sp)) == prefix)
            d = ((k >> jnp.uint32(sd)) & jnp.uint32(15)).astype(jnp.int32)
            cnts, last = plsc.scan_count(d, act)
            plsc.addupdate_scatter(hist_v, [d], cnts, mask=last & act)

    def select4(k_rem):
        M = hist_v[pl.ds(0, L)]
        rev = lax.rev(M, (0,))
        rc = plsc.cumsum(rev)
        istar = scalar(plsc.all_reduce_ffs(rc >= k_rem))
        sel = jnp.max(jnp.where(iota == istar, rc, 0))
        bincnt = jnp.max(jnp.where(iota == istar, rev, 0))
        digit = jnp.int32(L - 1) - istar
        return digit, k_rem - (sel - bincnt)

    wid = lax.axis_index("s") * NC + lax.axis_index("c")

    def row_body(j, carry):
        r = wid * RPW + j
        pltpu.sync_copy(sim_hbm.at[r], row_v)

        hist_full()
        d1, k_rem = select8(jnp.int32(KK))
        prefix = d1.astype(jnp.uint32)
        n = compact(prefix)

        for lvl in range(6):
            sd = 20 - 4 * lvl
            hist_cand(n, prefix, sd + 4, sd)
            dl, k_rem = select4(k_rem)
            prefix = (prefix << jnp.uint32(4)) | dl.astype(jnp.uint32)

        T = prefix
        m_take = k_rem

        trips = (n + L - 1) // L

        def wbody(i, st):
            cnt_eq, cnt_w = st
            ix = cidx_v[pl.ds(i * L, L)]
            k = keyat(ix)
            eq = ((i * L + iota) < n) & (k == T)
            eqi = eq.astype(jnp.int32)
            excl = plsc.cumsum(eqi) - eqi + cnt_eq
            win = eq & (excl < m_take)
            plsc.store_compressed(win_v.at[pl.ds(cnt_w, L)], ix, mask=win)
            cnt_eq = cnt_eq + scalar(plsc.all_reduce_population_count(eq))
            cnt_w = cnt_w + scalar(plsc.all_reduce_population_count(win))
            return (cnt_eq, cnt_w)

        lax.fori_loop(0, trips, wbody, (jnp.int32(0), jnp.int32(0)))

        @plsc.parallel_loop(0, NV, 1, unroll=8)
        def _mask(i):
            k = tokey(row_v[pl.ds(i * L, L)])
            row_v[pl.ds(i * L, L)] = jnp.where(
                k > T, jnp.float32(0), jnp.float32(jnp.inf))

        wtrips = (m_take + L - 1) // L

        def pbody(i, c):
            ixw = win_v[pl.ds(i * L, L)] & jnp.int32(NCOLS - 1)
            lanem = (i * L + iota) < m_take
            plsc.store_scatter(row_v, [ixw], zerosf, mask=lanem)
            return c

        lax.fori_loop(0, wtrips, pbody, jnp.int32(0))

        pltpu.sync_copy(row_v, out_hbm.at[r])
        return carry

    lax.fori_loop(0, RPW, row_body, jnp.int32(0))


_sc_kernel = functools.partial(
    pl.kernel,
    out_type=jax.ShapeDtypeStruct((NROWS, NCOLS), jnp.float32),
    mesh=_mesh,
    compiler_params=pltpu.CompilerParams(needs_layout_passes=False),
    scratch_types=[
        pltpu.VMEM((NCOLS,), jnp.float32),
        pltpu.VMEM((CCAP,), jnp.int32),
        pltpu.VMEM((NB,), jnp.int32),
        pltpu.VMEM((NV + L,), jnp.int32),
        pltpu.VMEM((WCAP,), jnp.int32),
    ],
)(_sc_body)


def kernel(sim):
    return _sc_kernel(sim)


# speedup vs baseline: 1.0185x; 1.0175x over previous
import functools

import jax
import jax.numpy as jnp
from jax import lax
from jax.experimental import pallas as pl
from jax.experimental.pallas import tpu as pltpu
from jax.experimental.pallas import tpu_sc as plsc

KK = 256
NROWS = 128
NCOLS = 32768
NC, NS, L = 2, 16, 16
NW = NC * NS
RPW = NROWS // NW
NV = NCOLS // L
NG = NV // L
NB = 256
CCAP = NCOLS + 16
WCAP = KK + 16

_mesh = plsc.VectorSubcoreMesh(core_axis_name="c", subcore_axis_name="s",
                               num_cores=NC, num_subcores=NS)


def _sc_body(sim_hbm, out_hbm, row_v, cidx_v, hist_v, pfx_v, win_v):
    iota = lax.iota(jnp.int32, L)
    ones = jnp.ones((L,), jnp.int32)
    zeros = jnp.zeros((L,), jnp.int32)
    zerosf = jnp.zeros((L,), jnp.float32)

    def tokey(v):
        b = lax.bitcast_convert_type(v, jnp.uint32)
        return jnp.where(b >= jnp.uint32(0x80000000), ~b,
                         b | jnp.uint32(0x80000000))

    def topdig(v):
        t = (lax.bitcast_convert_type(v, jnp.uint32) >> jnp.uint32(24)
             ).astype(jnp.int32)
        return jnp.where(t >= 128, 255 - t, t + 128)

    def keyat(ix):
        return tokey(plsc.load_gather(row_v, [ix & jnp.int32(NCOLS - 1)]))

    def scalar(x):
        return jnp.max(x) if x.ndim else x

    def hist_full():
        for g in range(L):
            hist_v[pl.ds(g * L, L)] = zeros

        @plsc.parallel_loop(0, NV, 1, unroll=8)
        def _hist(i):
            d = topdig(row_v[pl.ds(i * L, L)])
            cnts, last = plsc.scan_count(d)
            plsc.addupdate_scatter(hist_v, [d], cnts, mask=last)

    def select8(k_rem):
        def gbody(gr, st):
            carry, found, digit, above = st
            g = jnp.int32(L - 1) - gr
            M = hist_v[pl.ds(g * L, L)]
            revg = lax.rev(M, (0,))
            rcg = plsc.cumsum(revg) + carry
            ge = rcg >= k_rem
            anyge = jnp.max(ge.astype(jnp.int32))
            istar = scalar(plsc.all_reduce_ffs(ge))
            selrc = jnp.max(jnp.where(iota == istar, rcg, 0))
            selbin = jnp.max(jnp.where(iota == istar, revg, 0))
            hit = (anyge == 1) & (found == 0)
            digit = jnp.where(hit, g * L + (jnp.int32(L - 1) - istar), digit)
            above = jnp.where(hit, selrc - selbin, above)
            found = jnp.where(anyge == 1, jnp.int32(1), found)
            return (rcg[L - 1], found, digit, above)

        st = (jnp.int32(0), jnp.int32(0), jnp.int32(0), jnp.int32(0))
        _, _, digit, above = lax.fori_loop(0, L, gbody, st)
        return digit, k_rem - above

    def compact(d1):
        @plsc.parallel_loop(0, NG, 1, unroll=2)
        def _pa(i):
            acc = zeros
            for t in range(L):
                m = topdig(row_v[pl.ds((i * L + t) * L, L)]) == d1
                c = plsc.all_reduce_population_count(m)
                acc = jnp.where(iota == t, c, acc)
            pfx_v[pl.ds(i * L, L)] = acc

        def _pb(i, carry):
            c = pfx_v[pl.ds(i * L, L)]
            inc = plsc.cumsum(c)
            pfx_v[pl.ds(i * L, L)] = inc - c + carry
            return carry + inc[L - 1]

        n = lax.fori_loop(0, NG, _pb, jnp.int32(0))

        @plsc.parallel_loop(0, NV, 1, unroll=8)
        def _pc(i):
            m = topdig(row_v[pl.ds(i * L, L)]) == d1
            base = pfx_v[pl.ds(i, L)][0]
            plsc.store_compressed(cidx_v.at[pl.ds(base, L)], i * L + iota,
                                  mask=m)

        return n

    def hist_cand(n, prefix, sp, sd):
        hist_v[pl.ds(0, L)] = zeros
        trips = (n + L - 1) // L

        @plsc.parallel_loop(0, trips, 1, unroll=4)
        def _hc(i):
            k = keyat(cidx_v[pl.ds(i * L, L)])
            act = ((i * L + iota) < n) & ((k >> jnp.uint32(sp)) == prefix)
            d = ((k >> jnp.uint32(sd)) & jnp.uint32(15)).astype(jnp.int32)
            cnts, last = plsc.scan_count(d, act)
            plsc.addupdate_scatter(hist_v, [d], cnts, mask=last & act)

    def select4(k_rem):
        M = hist_v[pl.ds(0, L)]
        rev = lax.rev(M, (0,))
        rc = plsc.cumsum(rev)
        istar = scalar(plsc.all_reduce_ffs(rc >= k_rem))
        sel = jnp.max(jnp.where(iota == istar, rc, 0))
        bincnt = jnp.max(jnp.where(iota == istar, rev, 0))
        digit = jnp.int32(L - 1) - istar
        return digit, k_rem - (sel - bincnt)

    wid = lax.axis_index("s") * NC + lax.axis_index("c")

    def row_body(j, carry):
        r = wid * RPW + j
        pltpu.sync_copy(sim_hbm.at[r], row_v)

        hist_full()
        d1, k_rem = select8(jnp.int32(KK))
        prefix = d1.astype(jnp.uint32)
        n = compact(d1)

        for lvl in range(6):
            sd = 20 - 4 * lvl
            hist_cand(n, prefix, sd + 4, sd)
            dl, k_rem = select4(k_rem)
            prefix = (prefix << jnp.uint32(4)) | dl.astype(jnp.uint32)

        T = prefix
        m_take = k_rem

        trips = (n + L - 1) // L

        def wbody(i, st):
            cnt_eq, cnt_w = st
            ix = cidx_v[pl.ds(i * L, L)]
            k = keyat(ix)
            eq = ((i * L + iota) < n) & (k == T)
            eqi = eq.astype(jnp.int32)
            excl = plsc.cumsum(eqi) - eqi + cnt_eq
            win = eq & (excl < m_take)
            plsc.store_compressed(win_v.at[pl.ds(cnt_w, L)], ix, mask=win)
            cnt_eq = cnt_eq + scalar(plsc.all_reduce_population_count(eq))
            cnt_w = cnt_w + scalar(plsc.all_reduce_population_count(win))
            return (cnt_eq, cnt_w)

        lax.fori_loop(0, trips, wbody, (jnp.int32(0), jnp.int32(0)))

        @plsc.parallel_loop(0, NV, 1, unroll=8)
        def _mask(i):
            k = tokey(row_v[pl.ds(i * L, L)])
            row_v[pl.ds(i * L, L)] = jnp.where(
                k > T, jnp.float32(0), jnp.float32(jnp.inf))

        wtrips = (m_take + L - 1) // L

        def pbody(i, c):
            ixw = win_v[pl.ds(i * L, L)] & jnp.int32(NCOLS - 1)
            lanem = (i * L + iota) < m_take
            plsc.store_scatter(row_v, [ixw], zerosf, mask=lanem)
            return c

        lax.fori_loop(0, wtrips, pbody, jnp.int32(0))

        pltpu.sync_copy(row_v, out_hbm.at[r])
        return carry

    lax.fori_loop(0, RPW, row_body, jnp.int32(0))


_sc_kernel = functools.partial(
    pl.kernel,
    out_type=jax.ShapeDtypeStruct((NROWS, NCOLS), jnp.float32),
    mesh=_mesh,
    compiler_params=pltpu.CompilerParams(needs_layout_passes=False),
    scratch_types=[
        pltpu.VMEM((NCOLS,), jnp.float32),
        pltpu.VMEM((CCAP,), jnp.int32),
        pltpu.VMEM((NB,), jnp.int32),
        pltpu.VMEM((NV + L,), jnp.int32),
        pltpu.VMEM((WCAP,), jnp.int32),
    ],
)(_sc_body)


def kernel(sim):
    return _sc_kernel(sim)


# speedup vs baseline: 1.1139x; 1.0937x over previous
import functools

import jax
import jax.numpy as jnp
from jax import lax
from jax.experimental import pallas as pl
from jax.experimental.pallas import tpu as pltpu
from jax.experimental.pallas import tpu_sc as plsc

KK = 256
NROWS = 128
NCOLS = 32768
NC, NS, L = 2, 16, 16
NW = NC * NS
RPW = NROWS // NW
NV = NCOLS // L
NG = NV // L
NB = 256
CCAP = NCOLS + 16
WCAP = KK + 16

_mesh = plsc.VectorSubcoreMesh(core_axis_name="c", subcore_axis_name="s",
                               num_cores=NC, num_subcores=NS)


def _sc_body(sim_hbm, out_hbm, row_v, cidx_v, hist_v, pfx_v, win_v,
             sem_in, sem_out):
    iota = lax.iota(jnp.int32, L)
    ones = jnp.ones((L,), jnp.int32)
    zeros = jnp.zeros((L,), jnp.int32)
    zerosf = jnp.zeros((L,), jnp.float32)

    def tokey(v):
        b = lax.bitcast_convert_type(v, jnp.uint32)
        return jnp.where(b >= jnp.uint32(0x80000000), ~b,
                         b | jnp.uint32(0x80000000))

    def topdig(v):
        t = (lax.bitcast_convert_type(v, jnp.uint32) >> jnp.uint32(24)
             ).astype(jnp.int32)
        return jnp.where(t >= 128, 255 - t, t + 128)

    def keyat(rb, ix):
        return tokey(plsc.load_gather(
            row_v, [(ix & jnp.int32(NCOLS - 1)) + rb]))

    def scalar(x):
        return jnp.max(x) if x.ndim else x

    def hist_full(rb):
        for g in range(L):
            hist_v[pl.ds(g * L, L)] = zeros

        @plsc.parallel_loop(0, NV, 1, unroll=8)
        def _hist(i):
            d = topdig(row_v[pl.ds(rb + i * L, L)])
            cnts, last = plsc.scan_count(d)
            plsc.addupdate_scatter(hist_v, [d], cnts, mask=last)

    def select8(k_rem):
        def gbody(gr, st):
            carry, found, digit, above = st
            g = jnp.int32(L - 1) - gr
            M = hist_v[pl.ds(g * L, L)]
            revg = lax.rev(M, (0,))
            rcg = plsc.cumsum(revg) + carry
            ge = rcg >= k_rem
            anyge = jnp.max(ge.astype(jnp.int32))
            istar = scalar(plsc.all_reduce_ffs(ge))
            selrc = jnp.max(jnp.where(iota == istar, rcg, 0))
            selbin = jnp.max(jnp.where(iota == istar, revg, 0))
            hit = (anyge == 1) & (found == 0)
            digit = jnp.where(hit, g * L + (jnp.int32(L - 1) - istar), digit)
            above = jnp.where(hit, selrc - selbin, above)
            found = jnp.where(anyge == 1, jnp.int32(1), found)
            return (rcg[L - 1], found, digit, above)

        st = (jnp.int32(0), jnp.int32(0), jnp.int32(0), jnp.int32(0))
        _, _, digit, above = lax.fori_loop(0, L, gbody, st)
        return digit, k_rem - above

    def compact(rb, d1):
        @plsc.parallel_loop(0, NG, 1, unroll=2)
        def _pa(i):
            acc = zeros
            for t in range(L):
                m = topdig(row_v[pl.ds(rb + (i * L + t) * L, L)]) == d1
                c = plsc.all_reduce_population_count(m)
                acc = jnp.where(iota == t, c, acc)
            pfx_v[pl.ds(i * L, L)] = acc

        def _pb(i, carry):
            c = pfx_v[pl.ds(i * L, L)]
            inc = plsc.cumsum(c)
            pfx_v[pl.ds(i * L, L)] = inc - c + carry
            return carry + inc[L - 1]

        n = lax.fori_loop(0, NG, _pb, jnp.int32(0))

        @plsc.parallel_loop(0, NV, 1, unroll=8)
        def _pc(i):
            m = topdig(row_v[pl.ds(rb + i * L, L)]) == d1
            base = pfx_v[pl.ds(i, L)][0]
            plsc.store_compressed(cidx_v.at[pl.ds(base, L)], i * L + iota,
                                  mask=m)

        return n

    def hist_cand(rb, n, prefix, sp, sd):
        hist_v[pl.ds(0, L)] = zeros
        trips = (n + L - 1) // L

        @plsc.parallel_loop(0, trips, 1, unroll=4)
        def _hc(i):
            k = keyat(rb, cidx_v[pl.ds(i * L, L)])
            act = ((i * L + iota) < n) & ((k >> jnp.uint32(sp)) == prefix)
            d = ((k >> jnp.uint32(sd)) & jnp.uint32(15)).astype(jnp.int32)
            cnts, last = plsc.scan_count(d, act)
            plsc.addupdate_scatter(hist_v, [d], cnts, mask=last & act)

    def select4(k_rem):
        M = hist_v[pl.ds(0, L)]
        rev = lax.rev(M, (0,))
        rc = plsc.cumsum(rev)
        istar = scalar(plsc.all_reduce_ffs(rc >= k_rem))
        sel = jnp.max(jnp.where(iota == istar, rc, 0))
        bincnt = jnp.max(jnp.where(iota == istar, rev, 0))
        digit = jnp.int32(L - 1) - istar
        return digit, k_rem - (sel - bincnt)

    wid = lax.axis_index("s") * NC + lax.axis_index("c")

    def row_body(j, carry):
        sslot = j & 1
        rb = sslot * NCOLS
        r = wid * RPW + j
        dst = row_v.at[pl.ds(rb, NCOLS)]
        pltpu.make_async_copy(sim_hbm.at[r], dst, sem_in).wait()

        hist_full(rb)

        @pl.when(j > 0)
        def _():
            pltpu.make_async_copy(dst, out_hbm.at[r], sem_out).wait()

        @pl.when(j < RPW - 1)
        def _():
            nb = (1 - sslot) * NCOLS
            pltpu.async_copy(sim_hbm.at[r + 1],
                             row_v.at[pl.ds(nb, NCOLS)], sem_in)

        d1, k_rem = select8(jnp.int32(KK))
        prefix = d1.astype(jnp.uint32)
        n = compact(rb, d1)

        for lvl in range(6):
            sd = 20 - 4 * lvl
            hist_cand(rb, n, prefix, sd + 4, sd)
            dl, k_rem = select4(k_rem)
            prefix = (prefix << jnp.uint32(4)) | dl.astype(jnp.uint32)

        T = prefix
        m_take = k_rem

        trips = (n + L - 1) // L

        def wbody(i, st):
            cnt_eq, cnt_w = st
            ix = cidx_v[pl.ds(i * L, L)]
            k = keyat(rb, ix)
            eq = ((i * L + iota) < n) & (k == T)
            eqi = eq.astype(jnp.int32)
            excl = plsc.cumsum(eqi) - eqi + cnt_eq
            win = eq & (excl < m_take)
            plsc.store_compressed(win_v.at[pl.ds(cnt_w, L)], ix, mask=win)
            cnt_eq = cnt_eq + scalar(plsc.all_reduce_population_count(eq))
            cnt_w = cnt_w + scalar(plsc.all_reduce_population_count(win))
            return (cnt_eq, cnt_w)

        lax.fori_loop(0, trips, wbody, (jnp.int32(0), jnp.int32(0)))

        @plsc.parallel_loop(0, NV, 1, unroll=8)
        def _mask(i):
            k = tokey(row_v[pl.ds(rb + i * L, L)])
            row_v[pl.ds(rb + i * L, L)] = jnp.where(
                k > T, jnp.float32(0), jnp.float32(jnp.inf))

        wtrips = (m_take + L - 1) // L

        def pbody(i, c):
            ixw = (win_v[pl.ds(i * L, L)] & jnp.int32(NCOLS - 1)) + rb
            lanem = (i * L + iota) < m_take
            plsc.store_scatter(row_v, [ixw], zerosf, mask=lanem)
            return c

        lax.fori_loop(0, wtrips, pbody, jnp.int32(0))

        pltpu.async_copy(dst, out_hbm.at[r], sem_out)
        return carry

    pltpu.async_copy(sim_hbm.at[wid * RPW], row_v.at[pl.ds(0, NCOLS)],
                     sem_in)
    lax.fori_loop(0, RPW, row_body, jnp.int32(0))
    pltpu.make_async_copy(sim_hbm.at[wid * RPW],
                          row_v.at[pl.ds(0, NCOLS)], sem_out).wait()


_sc_kernel = functools.partial(
    pl.kernel,
    out_type=jax.ShapeDtypeStruct((NROWS, NCOLS), jnp.float32),
    mesh=_mesh,
    compiler_params=pltpu.CompilerParams(needs_layout_passes=False),
    scratch_types=[
        pltpu.VMEM((2 * NCOLS,), jnp.float32),
        pltpu.VMEM((CCAP,), jnp.int32),
        pltpu.VMEM((NB,), jnp.int32),
        pltpu.VMEM((NV + L,), jnp.int32),
        pltpu.VMEM((WCAP,), jnp.int32),
        pltpu.SemaphoreType.DMA,
        pltpu.SemaphoreType.DMA,
    ],
)(_sc_body)


def kernel(sim):
    return _sc_kernel(sim)
